# speedup vs baseline: 4.7477x; 4.7477x over previous
import jax
import jax.numpy as jnp
from jax.experimental import pallas as pl
from jax.experimental.pallas import tpu as pltpu

DIM = 4096
NUM_EXPERTS = 64
TOP_K = 8
TOKENS = 16384

BT = 512


def _router_body(x_ref, w_ref, b_ref, n_ref, out_ref, idx_ref):
    acc = jnp.dot(x_ref[...], w_ref[...], preferred_element_type=jnp.float32)
    acc = acc + b_ref[...]
    logits = acc[:, :NUM_EXPERTS]
    nl = acc[:, NUM_EXPERTS:]
    sp = jnp.maximum(nl, 0.0) + jnp.log1p(jnp.exp(-jnp.abs(nl)))
    noisy = logits + n_ref[...] * sp

    iota = jax.lax.broadcasted_iota(jnp.int32, (BT, NUM_EXPERTS), 1)
    work = noisy
    vals = []
    idxs = []
    for _ in range(TOP_K):
        m = jnp.max(work, axis=1, keepdims=True)
        is_m = work == m
        idx = jnp.min(jnp.where(is_m, iota, NUM_EXPERTS), axis=1, keepdims=True)
        vals.append(m)
        idxs.append(idx)
        work = jnp.where(iota == idx, -jnp.inf, work)

    v0 = vals[0]
    ps = [jnp.exp(v - v0) for v in vals]
    denom = ps[0]
    for p in ps[1:]:
        denom = denom + p
    out = jnp.zeros((BT, NUM_EXPERTS), jnp.float32)
    for p, idx in zip(ps, idxs):
        out = out + jnp.where(iota == idx, p / denom, 0.0)
    out_ref[...] = out
    idx_ref[...] = jnp.concatenate(idxs, axis=1)


@jax.jit
def kernel(x, W_route, b_route, W_noise, b_noise, noise):
    w = jnp.concatenate([W_route, W_noise], axis=0).T
    b = jnp.concatenate([b_route, b_noise])[None, :]
    grid = (TOKENS // BT,)
    out, idx = pl.pallas_call(
        _router_body,
        grid=grid,
        in_specs=[
            pl.BlockSpec((BT, DIM), lambda i: (i, 0)),
            pl.BlockSpec((DIM, 2 * NUM_EXPERTS), lambda i: (0, 0)),
            pl.BlockSpec((1, 2 * NUM_EXPERTS), lambda i: (0, 0)),
            pl.BlockSpec((BT, NUM_EXPERTS), lambda i: (i, 0)),
        ],
        out_specs=[
            pl.BlockSpec((BT, NUM_EXPERTS), lambda i: (i, 0)),
            pl.BlockSpec((BT, TOP_K), lambda i: (i, 0)),
        ],
        out_shape=[
            jax.ShapeDtypeStruct((TOKENS, NUM_EXPERTS), jnp.float32),
            jax.ShapeDtypeStruct((TOKENS, TOP_K), jnp.int32),
        ],
        compiler_params=pltpu.CompilerParams(
            dimension_semantics=("arbitrary",),
        ),
    )(x, w, b, noise)
    return (out, idx)


# speedup vs baseline: 5.2431x; 1.1043x over previous
import jax
import jax.numpy as jnp
from jax.experimental import pallas as pl
from jax.experimental.pallas import tpu as pltpu

DIM = 4096
NUM_EXPERTS = 64
TOP_K = 8
TOKENS = 16384

BT = 512


def _router_body(x_ref, w_ref, b_ref, n_ref, out_ref, idx_ref):
    acc = jnp.dot(x_ref[...], w_ref[...], preferred_element_type=jnp.float32)
    acc = acc + b_ref[...]
    logits = acc[:, :NUM_EXPERTS]
    nl = acc[:, NUM_EXPERTS:]
    sp = jnp.maximum(nl, 0.0) + jnp.log1p(jnp.exp(-jnp.abs(nl)))
    noisy = logits + n_ref[...] * sp

    iotaf = jax.lax.broadcasted_iota(
        jnp.int32, (BT, NUM_EXPERTS), 1).astype(jnp.float32)
    work = noisy
    vals = []
    idxs = []
    for _ in range(TOP_K):
        m = jnp.max(work, axis=1, keepdims=True)
        t = jnp.where(work == m, iotaf, jnp.float32(NUM_EXPERTS))
        idxf = jnp.min(t, axis=1, keepdims=True)
        vals.append(m)
        idxs.append(idxf)
        work = jnp.where(t == idxf, -jnp.inf, work)

    v0 = vals[0]
    ps = [jnp.exp(v - v0) for v in vals]
    denom = ps[0]
    for p in ps[1:]:
        denom = denom + p
    out = jnp.zeros((BT, NUM_EXPERTS), jnp.float32)
    for p, idxf in zip(ps, idxs):
        out = out + jnp.where(iotaf == idxf, p / denom, 0.0)
    out_ref[...] = out
    idx_ref[...] = jnp.concatenate(
        [idxf.astype(jnp.int32) for idxf in idxs], axis=1)


@jax.jit
def kernel(x, W_route, b_route, W_noise, b_noise, noise):
    w = jnp.concatenate([W_route, W_noise], axis=0).T
    b = jnp.concatenate([b_route, b_noise])[None, :]
    grid = (TOKENS // BT,)
    out, idx = pl.pallas_call(
        _router_body,
        grid=grid,
        in_specs=[
            pl.BlockSpec((BT, DIM), lambda i: (i, 0)),
            pl.BlockSpec((DIM, 2 * NUM_EXPERTS), lambda i: (0, 0)),
            pl.BlockSpec((1, 2 * NUM_EXPERTS), lambda i: (0, 0)),
            pl.BlockSpec((BT, NUM_EXPERTS), lambda i: (i, 0)),
        ],
        out_specs=[
            pl.BlockSpec((BT, NUM_EXPERTS), lambda i: (i, 0)),
            pl.BlockSpec((BT, TOP_K), lambda i: (i, 0)),
        ],
        out_shape=[
            jax.ShapeDtypeStruct((TOKENS, NUM_EXPERTS), jnp.float32),
            jax.ShapeDtypeStruct((TOKENS, TOP_K), jnp.int32),
        ],
        compiler_params=pltpu.CompilerParams(
            dimension_semantics=("arbitrary",),
        ),
    )(x, w, b, noise)
    return (out, idx)


# speedup vs baseline: 5.3762x; 1.0254x over previous
import jax
import jax.numpy as jnp
from jax.experimental import pallas as pl
from jax.experimental.pallas import tpu as pltpu

DIM = 4096
NUM_EXPERTS = 64
TOP_K = 8
TOKENS = 16384

BT = 512


def _router_body(x_ref, w_ref, b_ref, n_ref, out_ref, idx_ref):
    acc = jnp.dot(x_ref[...], w_ref[...], preferred_element_type=jnp.float32)
    acc = acc + b_ref[...]
    logits = acc[:, :NUM_EXPERTS]
    nl = acc[:, NUM_EXPERTS:]
    sp = jnp.maximum(nl, 0.0) + jnp.log1p(jnp.exp(-jnp.abs(nl)))
    noisy = logits + n_ref[...] * sp

    iotaf = jax.lax.broadcasted_iota(
        jnp.int32, (BT, NUM_EXPERTS), 1).astype(jnp.float32)
    work = noisy
    v0 = None
    idxs = []
    for k in range(TOP_K):
        m = jnp.max(work, axis=1, keepdims=True)
        if k == 0:
            v0 = m
        t = jnp.where(work == m, iotaf, jnp.float32(NUM_EXPERTS))
        idxf = jnp.min(t, axis=1, keepdims=True)
        idxs.append(idxf)
        work = jnp.where(t == idxf, -jnp.inf, work)

    sel = work == -jnp.inf
    p = jnp.where(sel, jnp.exp(noisy - v0), 0.0)
    denom = jnp.sum(p, axis=1, keepdims=True)
    out_ref[...] = p / denom
    idx_ref[...] = jnp.concatenate(idxs, axis=1).astype(jnp.int32)


@jax.jit
def kernel(x, W_route, b_route, W_noise, b_noise, noise):
    w = jnp.concatenate([W_route, W_noise], axis=0).T
    b = jnp.concatenate([b_route, b_noise])[None, :]
    grid = (TOKENS // BT,)
    out, idx = pl.pallas_call(
        _router_body,
        grid=grid,
        in_specs=[
            pl.BlockSpec((BT, DIM), lambda i: (i, 0)),
            pl.BlockSpec((DIM, 2 * NUM_EXPERTS), lambda i: (0, 0)),
            pl.BlockSpec((1, 2 * NUM_EXPERTS), lambda i: (0, 0)),
            pl.BlockSpec((BT, NUM_EXPERTS), lambda i: (i, 0)),
        ],
        out_specs=[
            pl.BlockSpec((BT, NUM_EXPERTS), lambda i: (i, 0)),
            pl.BlockSpec((BT, TOP_K), lambda i: (i, 0)),
        ],
        out_shape=[
            jax.ShapeDtypeStruct((TOKENS, NUM_EXPERTS), jnp.float32),
            jax.ShapeDtypeStruct((TOKENS, TOP_K), jnp.int32),
        ],
        compiler_params=pltpu.CompilerParams(
            dimension_semantics=("arbitrary",),
        ),
    )(x, w, b, noise)
    return (out, idx)


# speedup vs baseline: 5.7251x; 1.0649x over previous
import jax
import jax.numpy as jnp
from jax.experimental import pallas as pl
from jax.experimental.pallas import tpu as pltpu

DIM = 4096
NUM_EXPERTS = 64
TOP_K = 8
TOKENS = 16384

BT = 1024


def _router_body(x_ref, w_ref, b_ref, n_ref, out_ref, idx_ref):
    acc = jnp.dot(x_ref[...], w_ref[...], preferred_element_type=jnp.float32)
    acc = acc + b_ref[...]
    logits = acc[:, :NUM_EXPERTS]
    nl = acc[:, NUM_EXPERTS:]
    sp = jnp.maximum(nl, 0.0) + jnp.log1p(jnp.exp(-jnp.abs(nl)))
    noisy = logits + n_ref[...] * sp

    iotaf = jax.lax.broadcasted_iota(
        jnp.int32, (BT, NUM_EXPERTS), 1).astype(jnp.float32)
    work = noisy
    v0 = None
    idxs = []
    for k in range(TOP_K):
        m = jnp.max(work, axis=1, keepdims=True)
        if k == 0:
            v0 = m
        t = jnp.where(work == m, iotaf, jnp.float32(NUM_EXPERTS))
        idxf = jnp.min(t, axis=1, keepdims=True)
        idxs.append(idxf)
        work = jnp.where(t == idxf, -jnp.inf, work)

    sel = work == -jnp.inf
    p = jnp.where(sel, jnp.exp(noisy - v0), 0.0)
    denom = jnp.sum(p, axis=1, keepdims=True)
    out_ref[...] = p / denom
    idx_ref[...] = jnp.concatenate(idxs, axis=1).astype(jnp.int32)


@jax.jit
def kernel(x, W_route, b_route, W_noise, b_noise, noise):
    w = jnp.concatenate([W_route, W_noise], axis=0).T
    b = jnp.concatenate([b_route, b_noise])[None, :]
    grid = (TOKENS // BT,)
    out, idx = pl.pallas_call(
        _router_body,
        grid=grid,
        in_specs=[
            pl.BlockSpec((BT, DIM), lambda i: (i, 0)),
            pl.BlockSpec((DIM, 2 * NUM_EXPERTS), lambda i: (0, 0)),
            pl.BlockSpec((1, 2 * NUM_EXPERTS), lambda i: (0, 0)),
            pl.BlockSpec((BT, NUM_EXPERTS), lambda i: (i, 0)),
        ],
        out_specs=[
            pl.BlockSpec((BT, NUM_EXPERTS), lambda i: (i, 0)),
            pl.BlockSpec((BT, TOP_K), lambda i: (i, 0)),
        ],
        out_shape=[
            jax.ShapeDtypeStruct((TOKENS, NUM_EXPERTS), jnp.float32),
            jax.ShapeDtypeStruct((TOKENS, TOP_K), jnp.int32),
        ],
        compiler_params=pltpu.CompilerParams(
            dimension_semantics=("arbitrary",),
        ),
    )(x, w, b, noise)
    return (out, idx)
